# baseline (speedup 1.0000x reference)
import functools

import jax
import jax.numpy as jnp
from jax import lax
from jax.experimental import pallas as pl
from jax.experimental.pallas import tpu as pltpu
from jax.experimental.pallas import tpu_sc as plsc

D = 768
L = 16
NJ = D // L
C = 32
EPS = 1e-12


def _rsqrt_vec(x):
    i = plsc.bitcast(x, jnp.int32)
    i = jnp.int32(0x5F3759DF) - (i >> 1)
    y = plsc.bitcast(i, jnp.float32)
    for _ in range(3):
        y = y * (1.5 - 0.5 * x * y * y)
    return y


@functools.partial(jax.jit, static_argnames=("n_tokens", "seq_len"))
def _embed_ln(ids, tts, word_emb, pos_emb, type_emb, gamma, beta, *,
              n_tokens, seq_len):
    info = plsc.get_sparse_core_info()
    nw = info.num_cores * info.num_subcores
    n_per_w = n_tokens // nw
    n_chunks = n_per_w // C
    mesh = plsc.VectorSubcoreMesh(core_axis_name="c", subcore_axis_name="s")

    @functools.partial(
        pl.kernel,
        out_type=jax.ShapeDtypeStruct((n_tokens, D), jnp.float32),
        mesh=mesh,
        scratch_types=[
            pltpu.VMEM((C,), jnp.int32),
            pltpu.VMEM((C,), jnp.int32),
            pltpu.VMEM((C, D), jnp.float32),
            pltpu.VMEM((C, D), jnp.float32),
            pltpu.VMEM((C, D), jnp.float32),
            pltpu.VMEM((D,), jnp.float32),
            pltpu.VMEM((D,), jnp.float32),
            pltpu.SemaphoreType.DMA,
        ],
        compiler_params=pltpu.CompilerParams(needs_layout_passes=False),
    )
    def k(ids_hbm, tts_hbm, word_hbm, pos_hbm, type_hbm, gamma_hbm, beta_hbm,
          out_hbm, idx_v, tt_v, rows_v, pos_v, trow_v, gamma_v, beta_v, sem):
        wid = lax.axis_index("s") * info.num_cores + lax.axis_index("c")
        pltpu.sync_copy(gamma_hbm, gamma_v)
        pltpu.sync_copy(beta_hbm, beta_v)

        def chunk_body(kc, carry):
            base = wid * n_per_w + kc * C
            s0 = lax.rem(kc * C, seq_len)
            pltpu.sync_copy(ids_hbm.at[pl.ds(base, C)], idx_v)
            pltpu.sync_copy(tts_hbm.at[pl.ds(base, C)], tt_v)
            cp_w = pltpu.async_copy(word_hbm.at[idx_v], rows_v, sem)
            cp_t = pltpu.async_copy(type_hbm.at[tt_v], trow_v, sem)
            cp_p = pltpu.async_copy(pos_hbm.at[pl.ds(s0, C)], pos_v, sem)
            cp_w.wait()
            cp_t.wait()
            cp_p.wait()

            def token_body(i, c2):
                acc = jnp.zeros((L,), jnp.float32)
                acc2 = jnp.zeros((L,), jnp.float32)
                for j in range(NJ):
                    sl = pl.ds(j * L, L)
                    e = rows_v[i, sl] + pos_v[i, sl] + trow_v[i, sl]
                    rows_v[i, sl] = e
                    acc = acc + e
                    acc2 = acc2 + e * e
                s1 = jnp.sum(acc)
                s2 = jnp.sum(acc2)
                mean = s1 * (1.0 / D)
                var = s2 * (1.0 / D) - mean * mean
                inv = _rsqrt_vec(jnp.broadcast_to(var + EPS, (L,)))
                meanv = jnp.broadcast_to(mean, (L,))
                for j in range(NJ):
                    sl = pl.ds(j * L, L)
                    rows_v[i, sl] = ((rows_v[i, sl] - meanv) * inv
                                     * gamma_v[sl] + beta_v[sl])
                return c2

            lax.fori_loop(0, C, token_body, 0)
            pltpu.sync_copy(rows_v, out_hbm.at[pl.ds(base, C)])
            return carry

        lax.fori_loop(0, n_chunks, chunk_body, 0)

    return k(ids, tts, word_emb, pos_emb, type_emb, gamma, beta)


def kernel(input_ids, token_type_ids, attention_mask, word_emb, pos_emb,
           type_emb, gamma, beta):
    b, s = input_ids.shape
    out = _embed_ln(input_ids.reshape(-1), token_type_ids.reshape(-1),
                    word_emb, pos_emb, type_emb, gamma, beta,
                    n_tokens=b * s, seq_len=s)
    return out.reshape(b, s, D), attention_mask


# speedup vs baseline: 1.5576x; 1.5576x over previous
import functools

import jax
import jax.numpy as jnp
from jax import lax
from jax.experimental import pallas as pl
from jax.experimental.pallas import tpu as pltpu
from jax.experimental.pallas import tpu_sc as plsc

D = 768
L = 16
NJ = D // L
C = 16
NSLOT = 4
EPS = 1e-12


def _rsqrt_vec(x):
    i = plsc.bitcast(x, jnp.int32)
    i = jnp.int32(0x5F3759DF) - (i >> 1)
    y = plsc.bitcast(i, jnp.float32)
    for _ in range(3):
        y = y * (1.5 - 0.5 * x * y * y)
    return y


@functools.partial(jax.jit, static_argnames=("n_tokens", "seq_len"))
def _embed_ln(ids, tts, word_emb, pos_emb, type_emb, gamma, beta, *,
              n_tokens, seq_len):
    info = plsc.get_sparse_core_info()
    nw = info.num_cores * info.num_subcores
    n_per_w = n_tokens // nw
    n_chunks = n_per_w // C
    n_outer = n_chunks // NSLOT
    n_type = type_emb.shape[0]
    mesh = plsc.VectorSubcoreMesh(core_axis_name="c", subcore_axis_name="s")

    scratch = (
        [pltpu.VMEM((C, D), jnp.float32) for _ in range(NSLOT)]
        + [pltpu.VMEM((C, D), jnp.float32) for _ in range(NSLOT)]
        + [pltpu.VMEM((C,), jnp.int32) for _ in range(NSLOT)]
        + [pltpu.VMEM((C,), jnp.int32) for _ in range(NSLOT)]
        + [pltpu.VMEM((D,), jnp.float32),
           pltpu.VMEM((D,), jnp.float32),
           pltpu.VMEM((n_type, D), jnp.float32)]
        + [pltpu.SemaphoreType.DMA for _ in range(4 * NSLOT)]
    )

    @functools.partial(
        pl.kernel,
        out_type=jax.ShapeDtypeStruct((n_tokens, D), jnp.float32),
        mesh=mesh,
        scratch_types=scratch,
        compiler_params=pltpu.CompilerParams(needs_layout_passes=False),
    )
    def k(ids_hbm, tts_hbm, word_hbm, pos_hbm, type_hbm, gamma_hbm, beta_hbm,
          out_hbm, *sc):
        rows = sc[0:4]
        posb = sc[4:8]
        idsv = sc[8:12]
        ttv = sc[12:16]
        gamma_v, beta_v, type_v = sc[16], sc[17], sc[18]
        sem_ids = sc[19:23]
        sem_pos = sc[23:27]
        sem_w = sc[27:31]
        sem_o = sc[31:35]

        cid = lax.axis_index("c")
        sid = lax.axis_index("s")
        wid = sid * info.num_cores + cid

        pltpu.sync_copy(gamma_hbm, gamma_v)
        pltpu.sync_copy(beta_hbm, beta_v)
        pltpu.sync_copy(type_hbm, type_v)

        def base_of(kk):
            return wid * n_per_w + kk * C

        def issue_ids(kk, slot):
            pltpu.async_copy(ids_hbm.at[pl.ds(base_of(kk), C)], idsv[slot],
                             sem_ids[slot])
            pltpu.async_copy(tts_hbm.at[pl.ds(base_of(kk), C)], ttv[slot],
                             sem_ids[slot])

        def wait_ids(slot):
            pltpu.make_async_copy(ids_hbm.at[pl.ds(0, C)], idsv[slot],
                                  sem_ids[slot]).wait()
            pltpu.make_async_copy(tts_hbm.at[pl.ds(0, C)], ttv[slot],
                                  sem_ids[slot]).wait()

        def issue_pos(kk, slot):
            s0 = lax.rem(kk * C, seq_len)
            pltpu.async_copy(pos_hbm.at[pl.ds(s0, C)], posb[slot],
                             sem_pos[slot])

        def wait_pos(slot):
            pltpu.make_async_copy(pos_hbm.at[pl.ds(0, C)], posb[slot],
                                  sem_pos[slot]).wait()

        def issue_word(slot):
            pltpu.async_copy(word_hbm.at[idsv[slot]], rows[slot],
                             sem_w[slot])

        def wait_word(slot):
            pltpu.make_async_copy(word_hbm.at[idsv[slot]], rows[slot],
                                  sem_w[slot]).wait()

        def issue_out(kk, slot):
            pltpu.async_copy(rows[slot], out_hbm.at[pl.ds(base_of(kk), C)],
                             sem_o[slot])

        def wait_out(slot):
            pltpu.make_async_copy(rows[slot], out_hbm.at[pl.ds(0, C)],
                                  sem_o[slot]).wait()

        iota = lax.iota(jnp.int32, L)

        def compute(slot):
            rr = rows[slot]
            pp = posb[slot]
            tts_slot = ttv[slot]

            def token_body(i, c2):
                tt_splat = plsc.load_gather(
                    tts_slot, [jnp.broadcast_to(i, (L,)).astype(jnp.int32)])
                acc = jnp.zeros((L,), jnp.float32)
                acc2 = jnp.zeros((L,), jnp.float32)
                for j in range(NJ):
                    sl = pl.ds(j * L, L)
                    t = plsc.load_gather(type_v, [tt_splat, iota + (j * L)])
                    e = rr[i, sl] + pp[i, sl] + t
                    rr[i, sl] = e
                    acc = acc + e
                    acc2 = acc2 + e * e
                s1 = jnp.sum(acc)
                s2 = jnp.sum(acc2)
                mean = s1 * (1.0 / D)
                var = s2 * (1.0 / D) - mean * mean
                inv = _rsqrt_vec(jnp.broadcast_to(var + EPS, (L,)))
                meanv = jnp.broadcast_to(mean, (L,))
                for j in range(NJ):
                    sl = pl.ds(j * L, L)
                    rr[i, sl] = ((rr[i, sl] - meanv) * inv
                                 * gamma_v[sl] + beta_v[sl])
                return c2

            lax.fori_loop(0, C, token_body, 0)

        issue_ids(0, 0)
        issue_ids(1, 1)
        issue_ids(2, 2)
        issue_pos(0, 0)
        issue_pos(1, 1)
        wait_ids(0)
        issue_word(0)

        def outer(k0, carry):
            for p in range(NSLOT):
                kk = k0 * NSLOT + p
                sl1 = (p + 1) % NSLOT
                sl2 = (p + 2) % NSLOT
                sl3 = (p + 3) % NSLOT

                @pl.when(kk + 3 < n_chunks)
                def _():
                    issue_ids(kk + 3, sl3)

                @pl.when(kk >= 2)
                def _():
                    wait_out(sl2)

                @pl.when(kk + 2 < n_chunks)
                def _():
                    issue_pos(kk + 2, sl2)

                @pl.when(kk + 1 < n_chunks)
                def _():
                    wait_ids(sl1)
                    issue_word(sl1)

                wait_word(p)
                wait_pos(p)
                compute(p)
                issue_out(kk, p)
            return carry

        lax.fori_loop(0, n_outer, outer, 0)
        wait_out((n_chunks - 2) % NSLOT)
        wait_out((n_chunks - 1) % NSLOT)

    return k(ids, tts, word_emb, pos_emb, type_emb, gamma, beta)


def kernel(input_ids, token_type_ids, attention_mask, word_emb, pos_emb,
           type_emb, gamma, beta):
    b, s = input_ids.shape
    out = _embed_ln(input_ids.reshape(-1), token_type_ids.reshape(-1),
                    word_emb, pos_emb, type_emb, gamma, beta,
                    n_tokens=b * s, seq_len=s)
    return out.reshape(b, s, D), attention_mask
